# Initial kernel scaffold; baseline (speedup 1.0000x reference)
import jax
import jax.numpy as jnp
from jax import lax
from jax.experimental import pallas as pl
from jax.experimental.pallas import tpu as pltpu
from jax.experimental.pallas import tpu_sc as plsc

B = 16384
L = 50
D = 64
NC = 2
NS = 16
NW = NC * NS
BPW = B // NW
CB = 8
NCH = BPW // CB
ROWS_PER_STREAM = 100
STREAMS_PER_CHUNK = (CB * L) // ROWS_PER_STREAM


def _sc_scores_body(tgt_hbm, ctx_hbm, temb_hbm, cemb_hbm, scores_hbm,
                    tidx_v, trow_v, cidx_v, crow_v, score_v,
                    sem_t, sem_c):
    w = lax.axis_index("s") * NC + lax.axis_index("c")

    lane = lax.iota(jnp.int32, 16)
    m15 = lane == 15

    zeros16 = jnp.zeros((16,), jnp.float32)
    for b in range(CB):
        score_v[b, pl.ds(48, 16)] = zeros16

    pltpu.sync_copy(tgt_hbm.at[pl.ds(w * 4, 4)], tidx_v)
    tcps = [
        pltpu.async_copy(temb_hbm.at[tidx_v.at[k]],
                         trow_v.at[pl.ds(k * 128, 128)], sem_t)
        for k in range(4)
    ]
    for cp in tcps:
        cp.wait()

    def chunk_body(c, carry):
        pltpu.sync_copy(ctx_hbm.at[pl.ds(w * (BPW * L // ROWS_PER_STREAM)
                                         + c * STREAMS_PER_CHUNK,
                                         STREAMS_PER_CHUNK)], cidx_v)
        ccps = [
            pltpu.async_copy(cemb_hbm.at[cidx_v.at[k]],
                             crow_v.at[pl.ds(k * ROWS_PER_STREAM,
                                             ROWS_PER_STREAM)], sem_c)
            for k in range(STREAMS_PER_CHUNK)
        ]
        for cp in ccps:
            cp.wait()

        def batch_body(b, bc):
            wb = c * CB + b
            t0 = trow_v[wb, pl.ds(0, 16)]
            t1 = trow_v[wb, pl.ds(16, 16)]
            t2 = trow_v[wb, pl.ds(32, 16)]
            t3 = trow_v[wb, pl.ds(48, 16)]
            bvec = jnp.full((16,), b, jnp.int32)
            rb = b * L
            for l in range(L):
                r = rb + l
                c0 = crow_v[r, pl.ds(0, 16)]
                c1 = crow_v[r, pl.ds(16, 16)]
                c2 = crow_v[r, pl.ds(32, 16)]
                c3 = crow_v[r, pl.ds(48, 16)]
                m = (t0 * c0 + t1 * c1) + (t2 * c2 + t3 * c3)
                s = plsc.cumsum(m)
                plsc.store_scatter(
                    score_v,
                    [bvec, jnp.full((16,), l, jnp.int32)],
                    s, mask=m15)
            return bc

        lax.fori_loop(0, CB, batch_body, 0)
        pltpu.sync_copy(score_v,
                        scores_hbm.at[pl.ds(w * BPW + c * CB, CB)])
        return carry

    lax.fori_loop(0, NCH, chunk_body, 0)


def _sc_scores(tgt2, ctx2, temb, cemb):
    mesh = plsc.VectorSubcoreMesh(core_axis_name="c", subcore_axis_name="s")
    return pl.kernel(
        _sc_scores_body,
        out_type=jax.ShapeDtypeStruct((B, 64), jnp.float32),
        mesh=mesh,
        scratch_types=[
            pltpu.VMEM((4, 128), jnp.int32),
            pltpu.VMEM((BPW, D), jnp.float32),
            pltpu.VMEM((STREAMS_PER_CHUNK, ROWS_PER_STREAM), jnp.int32),
            pltpu.VMEM((CB * L, D), jnp.float32),
            pltpu.VMEM((CB, 64), jnp.float32),
            pltpu.SemaphoreType.DMA,
            pltpu.SemaphoreType.DMA,
        ],
    )(tgt2, ctx2, temb, cemb)


def _reduce_body(s_ref, o_ref):
    x = s_ref[...]
    col = lax.broadcasted_iota(jnp.int32, x.shape, 1)
    t = jnp.maximum(-x, 0.0) + jnp.log1p(jnp.exp(-jnp.abs(x)))
    o_ref[0, 0] = jnp.sum(jnp.where(col < L, t, 0.0)) * (1.0 / (B * L))


def _reduce(scores):
    return pl.pallas_call(
        _reduce_body,
        out_shape=jax.ShapeDtypeStruct((1, 1), jnp.float32),
        out_specs=pl.BlockSpec(memory_space=pltpu.SMEM),
    )(scores)


def kernel(target, context, target_embeddings, context_embeddings):
    tgt2 = target.reshape(B // 128, 128)
    ctx2 = context.reshape(B * L // ROWS_PER_STREAM, ROWS_PER_STREAM)
    scores = _sc_scores(tgt2, ctx2, target_embeddings, context_embeddings)
    return _reduce(scores)[0, 0]


# speedup vs baseline: 1.3821x; 1.3821x over previous
import jax
import jax.numpy as jnp
from jax import lax
from jax.experimental import pallas as pl
from jax.experimental.pallas import tpu as pltpu
from jax.experimental.pallas import tpu_sc as plsc

B = 16384
L = 50
D = 64
NC = 2
NS = 16
NW = NC * NS
BPW = B // NW
CB = 8
NCH = BPW // CB
ROWS_PER_STREAM = 100
STREAMS_PER_CHUNK = (CB * L) // ROWS_PER_STREAM


def _sc_scores_body(tgt_hbm, ctx_hbm, temb_hbm, cemb_hbm, scores_hbm,
                    tidx_v, trow_v, cidx_v, crow_v, score_v,
                    sem_t, sem_c):
    w = lax.axis_index("s") * NC + lax.axis_index("c")

    lane = lax.iota(jnp.int32, 16)
    m15 = lane == 15

    zeros16 = jnp.zeros((16,), jnp.float32)
    for b in range(CB):
        score_v[b, pl.ds(48, 16)] = zeros16

    pltpu.sync_copy(tgt_hbm.at[pl.ds(w * 4, 4)], tidx_v)
    tcps = [
        pltpu.async_copy(temb_hbm.at[tidx_v.at[k]],
                         trow_v.at[pl.ds(k * 128, 128)], sem_t)
        for k in range(4)
    ]
    for cp in tcps:
        cp.wait()

    def chunk_body(c, carry):
        pltpu.sync_copy(ctx_hbm.at[pl.ds(w * (BPW * L // ROWS_PER_STREAM)
                                         + c * STREAMS_PER_CHUNK,
                                         STREAMS_PER_CHUNK)], cidx_v)
        ccps = [
            pltpu.async_copy(cemb_hbm.at[cidx_v.at[k]],
                             crow_v.at[pl.ds(k * ROWS_PER_STREAM,
                                             ROWS_PER_STREAM)], sem_c)
            for k in range(STREAMS_PER_CHUNK)
        ]
        for cp in ccps:
            cp.wait()

        def batch_body(b, bc):
            wb = c * CB + b
            t0 = trow_v[wb, pl.ds(0, 16)]
            t1 = trow_v[wb, pl.ds(16, 16)]
            t2 = trow_v[wb, pl.ds(32, 16)]
            t3 = trow_v[wb, pl.ds(48, 16)]
            bvec = jnp.full((16,), b, jnp.int32)
            rb = b * L
            for l in range(L):
                r = rb + l
                c0 = crow_v[r, pl.ds(0, 16)]
                c1 = crow_v[r, pl.ds(16, 16)]
                c2 = crow_v[r, pl.ds(32, 16)]
                c3 = crow_v[r, pl.ds(48, 16)]
                m = (t0 * c0 + t1 * c1) + (t2 * c2 + t3 * c3)
                s = plsc.cumsum(m)
                plsc.store_scatter(
                    score_v,
                    [bvec, jnp.full((16,), l, jnp.int32)],
                    s, mask=m15)
            return bc

        lax.fori_loop(0, CB, batch_body, 0)
        pltpu.sync_copy(score_v,
                        scores_hbm.at[pl.ds(w * BPW + c * CB, CB)])
        return carry

    lax.fori_loop(0, NCH, chunk_body, 0)


def _sc_scores(tgt2, ctx2, temb, cemb):
    mesh = plsc.VectorSubcoreMesh(core_axis_name="c", subcore_axis_name="s")
    return pl.kernel(
        _sc_scores_body,
        out_type=jax.ShapeDtypeStruct((B, 64), jnp.float32),
        mesh=mesh,
        compiler_params=pltpu.CompilerParams(needs_layout_passes=False,
                                             use_tc_tiling_on_sc=False),
        scratch_types=[
            pltpu.VMEM((4, 128), jnp.int32),
            pltpu.VMEM((BPW, D), jnp.float32),
            pltpu.VMEM((STREAMS_PER_CHUNK, ROWS_PER_STREAM), jnp.int32),
            pltpu.VMEM((CB * L, D), jnp.float32),
            pltpu.VMEM((CB, 64), jnp.float32),
            pltpu.SemaphoreType.DMA,
            pltpu.SemaphoreType.DMA,
        ],
    )(tgt2, ctx2, temb, cemb)


def _reduce_body(s_ref, o_ref):
    x = s_ref[...]
    col = lax.broadcasted_iota(jnp.int32, x.shape, 1)
    t = jnp.maximum(-x, 0.0) + jnp.log1p(jnp.exp(-jnp.abs(x)))
    o_ref[0, 0] = jnp.sum(jnp.where(col < L, t, 0.0)) * (1.0 / (B * L))


def _reduce(scores):
    return pl.pallas_call(
        _reduce_body,
        out_shape=jax.ShapeDtypeStruct((1, 1), jnp.float32),
        out_specs=pl.BlockSpec(memory_space=pltpu.SMEM),
    )(scores)


def kernel(target, context, target_embeddings, context_embeddings):
    tgt2 = target.reshape(B // 128, 128)
    ctx2 = context.reshape(B * L // ROWS_PER_STREAM, ROWS_PER_STREAM)
    scores = _sc_scores(tgt2, ctx2, target_embeddings, context_embeddings)
    return _reduce(scores)[0, 0]


# speedup vs baseline: 1.7587x; 1.2725x over previous
import jax
import jax.numpy as jnp
from jax import lax
from jax.experimental import pallas as pl
from jax.experimental.pallas import tpu as pltpu
from jax.experimental.pallas import tpu_sc as plsc

B = 16384
L = 50
D = 64
NC = 2
NS = 16
NW = NC * NS
BPW = B // NW
CB = 4
NCH = BPW // CB
CROWS = CB * L
IDXR = 100
CIR = CROWS // IDXR
RPW = 256
_IDX_GROUPS = [(o, tuple(range(16))) for o in (0, 16, 32, 48, 64, 80)] + [
    (84, (12, 13, 14, 15))]


def _sc_scores_body(tgt_hbm, ctx_hbm, temb_hbm, cemb_hbm, scores_hbm,
                    tidx_v, trow_v, cidx_v, crow0, crow1, score_v,
                    sem_t, sem_c0, sem_c1):
    w = lax.axis_index("s") * NC + lax.axis_index("c")

    lane = lax.iota(jnp.int32, 16)
    m15 = lane == 15
    zeros16 = jnp.zeros((16,), jnp.float32)

    for r in range(CB):
        score_v[r, pl.ds(48, 16)] = zeros16

    def drain(dummy_src, dst, sem):
        pltpu.make_async_copy(dummy_src, dst, sem).wait()

    def issue_rows(idx_ref, nrows, rowlen, groups, table_hbm, dst_ref, sem):
        def go(k, c):
            for off, lanes in groups:
                iv = idx_ref[k, pl.ds(off, 16)]
                for u in lanes:
                    slot = off + u
                    pltpu.async_copy(
                        table_hbm.at[pl.ds(iv[u], 1)],
                        dst_ref.at[pl.ds(k * rowlen + slot, 1)], sem)
            return c
        lax.fori_loop(0, nrows, go, 0)

    _TGT_GROUPS = [(o, tuple(range(16))) for o in range(0, 128, 16)]

    pltpu.sync_copy(tgt_hbm.at[pl.ds(w * 4, 4)], tidx_v)
    issue_rows(tidx_v, 4, 128, _TGT_GROUPS, temb_hbm, trow_v, sem_t)
    drain(temb_hbm.at[pl.ds(0, BPW)], trow_v, sem_t)

    def idx_copy(ch):
        off = w * RPW + jnp.minimum(ch, NCH - 1) * CIR
        pltpu.sync_copy(ctx_hbm.at[pl.ds(off, CIR)], cidx_v)

    def crow_drain(crow, sem):
        drain(cemb_hbm.at[pl.ds(0, CROWS)], crow, sem)

    def compute(crow, ch):
        def bb(b, c):
            wb = ch * CB + b
            t0 = trow_v[wb, pl.ds(0, 16)]
            t1 = trow_v[wb, pl.ds(16, 16)]
            t2 = trow_v[wb, pl.ds(32, 16)]
            t3 = trow_v[wb, pl.ds(48, 16)]
            bvec = jnp.full((16,), b, jnp.int32)
            rb = b * L
            for l in range(L):
                r = rb + l
                c0 = crow[r, pl.ds(0, 16)]
                c1 = crow[r, pl.ds(16, 16)]
                c2 = crow[r, pl.ds(32, 16)]
                c3 = crow[r, pl.ds(48, 16)]
                m = (t0 * c0 + t1 * c1) + (t2 * c2 + t3 * c3)
                s = plsc.cumsum(m)
                plsc.store_scatter(
                    score_v, [bvec, jnp.full((16,), l, jnp.int32)],
                    s, mask=m15)
            return c
        lax.fori_loop(0, CB, bb, 0)
        pltpu.sync_copy(score_v,
                        scores_hbm.at[pl.ds(w * BPW + ch * CB, CB)])

    idx_copy(jnp.int32(0))
    issue_rows(cidx_v, CIR, IDXR, _IDX_GROUPS, cemb_hbm, crow0, sem_c0)

    def two_chunks(i, c):
        ch = i * 2
        idx_copy(ch + 1)
        issue_rows(cidx_v, CIR, IDXR, _IDX_GROUPS, cemb_hbm, crow1, sem_c1)
        crow_drain(crow0, sem_c0)
        compute(crow0, ch)
        idx_copy(ch + 2)
        issue_rows(cidx_v, CIR, IDXR, _IDX_GROUPS, cemb_hbm, crow0, sem_c0)
        crow_drain(crow1, sem_c1)
        compute(crow1, ch + 1)
        return c

    lax.fori_loop(0, NCH // 2, two_chunks, 0)

    crow_drain(crow0, sem_c0)


def _sc_scores(tgt2, ctx2, temb, cemb):
    mesh = plsc.VectorSubcoreMesh(core_axis_name="c", subcore_axis_name="s")
    return pl.kernel(
        _sc_scores_body,
        out_type=jax.ShapeDtypeStruct((B, 64), jnp.float32),
        mesh=mesh,
        compiler_params=pltpu.CompilerParams(needs_layout_passes=False),
        scratch_types=[
            pltpu.VMEM((4, 128), jnp.int32),
            pltpu.VMEM((BPW, D), jnp.float32),
            pltpu.VMEM((CIR, IDXR), jnp.int32),
            pltpu.VMEM((CROWS, D), jnp.float32),
            pltpu.VMEM((CROWS, D), jnp.float32),
            pltpu.VMEM((CB, 64), jnp.float32),
            pltpu.SemaphoreType.DMA,
            pltpu.SemaphoreType.DMA,
            pltpu.SemaphoreType.DMA,
        ],
    )(tgt2, ctx2, temb, cemb)


def _reduce_body(s_ref, o_ref):
    x = s_ref[...]
    col = lax.broadcasted_iota(jnp.int32, x.shape, 1)
    t = jnp.maximum(-x, 0.0) + jnp.log1p(jnp.exp(-jnp.abs(x)))
    o_ref[0, 0] = jnp.sum(jnp.where(col < L, t, 0.0)) * (1.0 / (B * L))


def _reduce(scores):
    return pl.pallas_call(
        _reduce_body,
        out_shape=jax.ShapeDtypeStruct((1, 1), jnp.float32),
        out_specs=pl.BlockSpec(memory_space=pltpu.SMEM),
    )(scores)


def kernel(target, context, target_embeddings, context_embeddings):
    tgt2 = target.reshape(B // 128, 128)
    ctx2 = context.reshape(B * L // IDXR, IDXR)
    scores = _sc_scores(tgt2, ctx2, target_embeddings, context_embeddings)
    return _reduce(scores)[0, 0]
